# baseline (speedup 1.0000x reference)
import functools

import jax
import jax.numpy as jnp
from jax import lax
from jax.experimental import pallas as pl
from jax.experimental.pallas import tpu as pltpu
from jax.experimental.pallas import tpu_sc as plsc

N_ROWS = 16384
N_ATOMS = 200
NUM_WORKERS = 32
ROWS_PER_W = N_ROWS // NUM_WORKERS
CHUNK_ROWS = 64
NUM_CHUNKS = ROWS_PER_W // CHUNK_ROWS
CHUNK_WORDS = CHUNK_ROWS * N_ATOMS
LANES = 16


def _sc_body(species_hbm, energies_hbm, tbl_hbm, out_hbm,
             sp_v, tbl_v, en_v, row_v):
    wid = lax.axis_index("s") * 2 + lax.axis_index("c")
    base_row = wid * ROWS_PER_W

    pltpu.sync_copy(tbl_hbm, tbl_v)
    pltpu.sync_copy(energies_hbm.at[pl.ds(base_row, ROWS_PER_W)], en_v)
    lane = lax.iota(jnp.int32, LANES)

    for c in range(NUM_CHUNKS):
        chunk_row0 = base_row + c * CHUNK_ROWS
        pltpu.sync_copy(
            species_hbm.at[pl.ds(chunk_row0 * N_ATOMS, CHUNK_WORDS)],
            sp_v.at[pl.ds(0, CHUNK_WORDS)])

        def row_body(r, _):
            rb = r * N_ATOMS
            acc = jnp.zeros((LANES,), jnp.float32)
            for g in range(6):
                va = sp_v[pl.ds(rb + 32 * g, LANES)]
                vb = sp_v[pl.ds(rb + 32 * g + LANES, LANES)]
                acc = acc + plsc.load_gather(tbl_v, [va + vb * 8])
            vr = sp_v[pl.ds(rb + 192, LANES)]
            idx2 = jnp.where(lane < 8, vr + 64, 72)
            acc = acc + plsc.load_gather(tbl_v, [idx2])
            s = jnp.sum(acc)
            plsc.store_scatter(row_v, [jnp.full((LANES,), r, jnp.int32)],
                               jnp.full((LANES,), s, jnp.float32),
                               mask=lane == 0)
            return 0

        lax.fori_loop(0, CHUNK_ROWS, row_body, 0)

        for k in range(CHUNK_ROWS // LANES):
            rv = row_v[pl.ds(k * LANES, LANES)]
            ev = en_v[pl.ds(c * CHUNK_ROWS + k * LANES, LANES)]
            row_v[pl.ds(k * LANES, LANES)] = rv + ev
        pltpu.sync_copy(row_v,
                        out_hbm.at[pl.ds(chunk_row0, CHUNK_ROWS)])


@jax.jit
def _run(species_flat, energies, tbl):
    mesh = plsc.VectorSubcoreMesh(core_axis_name="c", subcore_axis_name="s")
    return pl.kernel(
        _sc_body,
        mesh=mesh,
        compiler_params=pltpu.CompilerParams(needs_layout_passes=False),
        out_type=jax.ShapeDtypeStruct((N_ROWS,), jnp.float32),
        scratch_types=[
            pltpu.VMEM((CHUNK_WORDS + LANES,), jnp.int32),
            pltpu.VMEM((80,), jnp.float32),
            pltpu.VMEM((ROWS_PER_W,), jnp.float32),
            pltpu.VMEM((CHUNK_ROWS,), jnp.float32),
        ],
    )(species_flat, energies, tbl)


def kernel(species, energies, self_energies):
    se = self_energies.astype(jnp.float32)
    pair = (se[:, None] + se[None, :]).reshape(64)
    tbl = jnp.concatenate([pair, se, jnp.zeros((8,), jnp.float32)])
    shifted = _run(species.reshape(-1), energies, tbl)
    return (species, shifted)


# speedup vs baseline: 1.0163x; 1.0163x over previous
import functools

import jax
import jax.numpy as jnp
from jax import lax
from jax.experimental import pallas as pl
from jax.experimental.pallas import tpu as pltpu
from jax.experimental.pallas import tpu_sc as plsc

N_ROWS = 16384
N_ATOMS = 200
NUM_WORKERS = 32
ROWS_PER_W = N_ROWS // NUM_WORKERS
CHUNK_ROWS = 64
NUM_CHUNKS = ROWS_PER_W // CHUNK_ROWS
CHUNK_WORDS = CHUNK_ROWS * N_ATOMS
LANES = 16


def _sc_body(species_hbm, energies_hbm, tbl_hbm, out_hbm,
             sp_v, tbl_v, en_v, row_v):
    wid = lax.axis_index("s") * 2 + lax.axis_index("c")
    base_row = wid * ROWS_PER_W

    pltpu.sync_copy(tbl_hbm, tbl_v)
    pltpu.sync_copy(energies_hbm.at[pl.ds(base_row, ROWS_PER_W)], en_v)
    lane = lax.iota(jnp.int32, LANES)

    for c in range(NUM_CHUNKS):
        chunk_row0 = base_row + c * CHUNK_ROWS
        pltpu.sync_copy(
            species_hbm.at[pl.ds(chunk_row0 * N_ATOMS, CHUNK_WORDS)],
            sp_v.at[pl.ds(0, CHUNK_WORDS)])

        @plsc.parallel_loop(0, CHUNK_ROWS, unroll=8)
        def row_body(r):
            rb = r * N_ATOMS
            acc0 = jnp.zeros((LANES,), jnp.float32)
            acc1 = jnp.zeros((LANES,), jnp.float32)
            for g in range(6):
                va = sp_v[pl.ds(rb + 32 * g, LANES)]
                vb = sp_v[pl.ds(rb + 32 * g + LANES, LANES)]
                gathered = plsc.load_gather(tbl_v, [va + vb * 8])
                if g % 2 == 0:
                    acc0 = acc0 + gathered
                else:
                    acc1 = acc1 + gathered
            vr = sp_v[pl.ds(rb + 192, LANES)]
            idx2 = jnp.where(lane < 8, vr + 64, 72)
            acc1 = acc1 + plsc.load_gather(tbl_v, [idx2])
            s = jnp.sum(acc0 + acc1)
            plsc.store_scatter(row_v, [jnp.full((LANES,), r, jnp.int32)],
                               jnp.full((LANES,), s, jnp.float32),
                               mask=lane == 0)

        for k in range(CHUNK_ROWS // LANES):
            rv = row_v[pl.ds(k * LANES, LANES)]
            ev = en_v[pl.ds(c * CHUNK_ROWS + k * LANES, LANES)]
            row_v[pl.ds(k * LANES, LANES)] = rv + ev
        pltpu.sync_copy(row_v,
                        out_hbm.at[pl.ds(chunk_row0, CHUNK_ROWS)])


@jax.jit
def _run(species_flat, energies, tbl):
    mesh = plsc.VectorSubcoreMesh(core_axis_name="c", subcore_axis_name="s")
    return pl.kernel(
        _sc_body,
        mesh=mesh,
        compiler_params=pltpu.CompilerParams(needs_layout_passes=False),
        out_type=jax.ShapeDtypeStruct((N_ROWS,), jnp.float32),
        scratch_types=[
            pltpu.VMEM((CHUNK_WORDS + LANES,), jnp.int32),
            pltpu.VMEM((80,), jnp.float32),
            pltpu.VMEM((ROWS_PER_W,), jnp.float32),
            pltpu.VMEM((CHUNK_ROWS,), jnp.float32),
        ],
    )(species_flat, energies, tbl)


def kernel(species, energies, self_energies):
    se = self_energies.astype(jnp.float32)
    pair = (se[:, None] + se[None, :]).reshape(64)
    tbl = jnp.concatenate([pair, se, jnp.zeros((8,), jnp.float32)])
    shifted = _run(species.reshape(-1), energies, tbl)
    return (species, shifted)


# speedup vs baseline: 1.3809x; 1.3587x over previous
import functools

import jax
import jax.numpy as jnp
from jax import lax
from jax.experimental import pallas as pl
from jax.experimental.pallas import tpu as pltpu
from jax.experimental.pallas import tpu_sc as plsc

N_ROWS = 16384
N_ATOMS = 200
NUM_WORKERS = 32
ROWS_PER_W = N_ROWS // NUM_WORKERS
CHUNK_ROWS = 64
NUM_CHUNKS = ROWS_PER_W // CHUNK_ROWS
LANES = 16


def _sc_body(species_hbm, energies_hbm, tbl_hbm, out_hbm,
             sp_v, tbl_v, en_v, row_v):
    wid = lax.axis_index("s") * 2 + lax.axis_index("c")
    base_row = wid * ROWS_PER_W

    pltpu.sync_copy(tbl_hbm, tbl_v)
    pltpu.sync_copy(energies_hbm.at[pl.ds(base_row, ROWS_PER_W)], en_v)
    lane = lax.iota(jnp.int32, LANES)

    for c in range(NUM_CHUNKS):
        chunk_row0 = base_row + c * CHUNK_ROWS
        pltpu.sync_copy(species_hbm.at[pl.ds(chunk_row0, CHUNK_ROWS)], sp_v)

        @plsc.parallel_loop(0, CHUNK_ROWS, unroll=8)
        def row_body(r):
            acc0 = jnp.zeros((LANES,), jnp.float32)
            acc1 = jnp.zeros((LANES,), jnp.float32)
            for g in range(5):
                va = sp_v[r, pl.ds(32 * g, LANES)]
                vb = sp_v[r, pl.ds(32 * g + LANES, LANES)]
                gathered = plsc.load_gather(tbl_v, [va + vb * 8])
                if g % 2 == 0:
                    acc0 = acc0 + gathered
                else:
                    acc1 = acc1 + gathered
            vr = sp_v[r, pl.ds(160, LANES)]
            idx2 = jnp.where(lane < 8, vr + 64, 72)
            acc0 = acc0 + plsc.load_gather(tbl_v, [idx2])
            va = sp_v[r, pl.ds(168, LANES)]
            vb = sp_v[r, pl.ds(184, LANES)]
            acc1 = acc1 + plsc.load_gather(tbl_v, [va + vb * 8])
            s = jnp.sum(acc0 + acc1)
            plsc.store_scatter(row_v, [jnp.full((LANES,), r, jnp.int32)],
                               jnp.full((LANES,), s, jnp.float32),
                               mask=lane == 0)

        for k in range(CHUNK_ROWS // LANES):
            rv = row_v[pl.ds(k * LANES, LANES)]
            ev = en_v[pl.ds(c * CHUNK_ROWS + k * LANES, LANES)]
            row_v[pl.ds(k * LANES, LANES)] = rv + ev
        pltpu.sync_copy(row_v,
                        out_hbm.at[pl.ds(chunk_row0, CHUNK_ROWS)])


@jax.jit
def _run(species, energies, tbl):
    mesh = plsc.VectorSubcoreMesh(core_axis_name="c", subcore_axis_name="s")
    return pl.kernel(
        _sc_body,
        mesh=mesh,
        compiler_params=pltpu.CompilerParams(needs_layout_passes=False),
        out_type=jax.ShapeDtypeStruct((N_ROWS,), jnp.float32),
        scratch_types=[
            pltpu.VMEM((CHUNK_ROWS, N_ATOMS), jnp.int32),
            pltpu.VMEM((80,), jnp.float32),
            pltpu.VMEM((ROWS_PER_W,), jnp.float32),
            pltpu.VMEM((CHUNK_ROWS,), jnp.float32),
        ],
    )(species, energies, tbl)


def kernel(species, energies, self_energies):
    se = self_energies.astype(jnp.float32)
    pair = (se[:, None] + se[None, :]).reshape(64)
    tbl = jnp.concatenate([pair, se, jnp.zeros((8,), jnp.float32)])
    shifted = _run(species, energies, tbl)
    return (species, shifted)


# speedup vs baseline: 2.2147x; 1.6038x over previous
import functools

import jax
import jax.numpy as jnp
from jax import lax
from jax.experimental import pallas as pl
from jax.experimental.pallas import tpu as pltpu
from jax.experimental.pallas import tpu_sc as plsc

N_ROWS = 16384
N_ATOMS = 200
NUM_WORKERS = 32
COLS_PER_W = N_ROWS // NUM_WORKERS
CHUNK_COLS = 256
NUM_CHUNKS = COLS_PER_W // CHUNK_COLS
LANES = 16
GROUPS = CHUNK_COLS // LANES


def _sc_body(spT_hbm, energies_hbm, tbl_hbm, out_spT_hbm, out_hbm,
             sp0_v, sp1_v, tbl_v, en_v, row_v,
             sem_in0, sem_in1, sem_out0, sem_out1, sem_wb):
    wid = lax.axis_index("s") * 2 + lax.axis_index("c")
    base_col = wid * COLS_PER_W

    pltpu.sync_copy(tbl_hbm, tbl_v)
    pltpu.sync_copy(energies_hbm.at[pl.ds(base_col, COLS_PER_W)], en_v)

    in0 = pltpu.async_copy(
        spT_hbm.at[:, pl.ds(base_col, CHUNK_COLS)], sp0_v, sem_in0)
    in1 = pltpu.async_copy(
        spT_hbm.at[:, pl.ds(base_col + CHUNK_COLS, CHUNK_COLS)], sp1_v,
        sem_in1)

    for c, (sp_v, cin, sem_out) in enumerate(
            ((sp0_v, in0, sem_out0), (sp1_v, in1, sem_out1))):
        col0 = base_col + c * CHUNK_COLS
        cin.wait()
        wb = pltpu.async_copy(sp_v, out_spT_hbm.at[:, pl.ds(col0, CHUNK_COLS)],
                              sem_out)

        @plsc.parallel_loop(0, GROUPS, unroll=2)
        def group_body(g):
            lb = g * LANES
            acc0 = jnp.zeros((LANES,), jnp.float32)
            acc1 = jnp.zeros((LANES,), jnp.float32)
            for q in range(N_ATOMS // 4):
                a = 4 * q
                s0 = sp_v[a, pl.ds(lb, LANES)]
                s1 = sp_v[a + 1, pl.ds(lb, LANES)]
                s2 = sp_v[a + 2, pl.ds(lb, LANES)]
                s3 = sp_v[a + 3, pl.ds(lb, LANES)]
                idx = s0 + s1 * 8 + s2 * 64 + s3 * 512
                gathered = plsc.load_gather(tbl_v, [idx])
                if q % 2 == 0:
                    acc0 = acc0 + gathered
                else:
                    acc1 = acc1 + gathered
            ev = en_v[pl.ds(c * CHUNK_COLS + lb, LANES)]
            row_v[pl.ds(lb, LANES)] = acc0 + acc1 + ev

        pltpu.sync_copy(row_v, out_hbm.at[pl.ds(col0, CHUNK_COLS)])
        wb.wait()


@jax.jit
def _run(spT, energies, tbl):
    mesh = plsc.VectorSubcoreMesh(core_axis_name="c", subcore_axis_name="s")
    return pl.kernel(
        _sc_body,
        mesh=mesh,
        compiler_params=pltpu.CompilerParams(needs_layout_passes=False),
        out_type=(jax.ShapeDtypeStruct((N_ATOMS, N_ROWS), jnp.int32),
                  jax.ShapeDtypeStruct((N_ROWS,), jnp.float32)),
        scratch_types=[
            pltpu.VMEM((N_ATOMS, CHUNK_COLS), jnp.int32),
            pltpu.VMEM((N_ATOMS, CHUNK_COLS), jnp.int32),
            pltpu.VMEM((4096,), jnp.float32),
            pltpu.VMEM((COLS_PER_W,), jnp.float32),
            pltpu.VMEM((CHUNK_COLS,), jnp.float32),
            pltpu.SemaphoreType.DMA,
            pltpu.SemaphoreType.DMA,
            pltpu.SemaphoreType.DMA,
            pltpu.SemaphoreType.DMA,
            pltpu.SemaphoreType.DMA,
        ],
    )(spT, energies, tbl)


def kernel(species, energies, self_energies):
    se = self_energies.astype(jnp.float32)
    tbl = (se[:, None, None, None] + se[None, :, None, None]
           + se[None, None, :, None] + se[None, None, None, :]).reshape(4096)
    out_spT, shifted = _run(species.T, energies, tbl)
    return (out_spT.T, shifted)


# speedup vs baseline: 2.6066x; 1.1769x over previous
import functools

import jax
import jax.numpy as jnp
from jax import lax
from jax.experimental import pallas as pl
from jax.experimental.pallas import tpu as pltpu
from jax.experimental.pallas import tpu_sc as plsc

N_ROWS = 16384
N_ATOMS = 200
NUM_WORKERS = 32
COLS_PER_W = N_ROWS // NUM_WORKERS
CHUNK_COLS = 256
NUM_CHUNKS = COLS_PER_W // CHUNK_COLS
LANES = 16
GROUPS = CHUNK_COLS // LANES


def _sc_body(spT_hbm, energies_hbm, tbl_hbm, out_spT_hbm, out_hbm,
             sp0_v, sp1_v, tbl_v, en_v, row_v,
             sem_in0, sem_in1, sem_out0, sem_out1, sem_wb):
    wid = lax.axis_index("s") * 2 + lax.axis_index("c")
    base_col = wid * COLS_PER_W

    pltpu.sync_copy(tbl_hbm, tbl_v)
    pltpu.sync_copy(energies_hbm.at[pl.ds(base_col, COLS_PER_W)], en_v)

    in0 = pltpu.async_copy(
        spT_hbm.at[:, pl.ds(base_col, CHUNK_COLS)], sp0_v, sem_in0)
    in1 = pltpu.async_copy(
        spT_hbm.at[:, pl.ds(base_col + CHUNK_COLS, CHUNK_COLS)], sp1_v,
        sem_in1)

    for c, (sp_v, cin, sem_out) in enumerate(
            ((sp0_v, in0, sem_out0), (sp1_v, in1, sem_out1))):
        col0 = base_col + c * CHUNK_COLS
        cin.wait()
        wb = pltpu.async_copy(sp_v, out_spT_hbm.at[:, pl.ds(col0, CHUNK_COLS)],
                              sem_out)

        @plsc.parallel_loop(0, GROUPS, unroll=1)
        def group_body(g):
            lb = g * LANES
            zero = jnp.zeros((LANES,), jnp.float32)

            @plsc.parallel_loop(0, N_ATOMS // 8, unroll=5,
                                carry=(zero, zero))
            def quad_pair(qi, accs):
                acc0, acc1 = accs
                a = 8 * qi
                for j, _ in enumerate(accs):
                    b = a + 4 * j
                    s0 = sp_v[b, pl.ds(lb, LANES)]
                    s1 = sp_v[b + 1, pl.ds(lb, LANES)]
                    s2 = sp_v[b + 2, pl.ds(lb, LANES)]
                    s3 = sp_v[b + 3, pl.ds(lb, LANES)]
                    idx = s0 + s1 * 8 + s2 * 64 + s3 * 512
                    gathered = plsc.load_gather(tbl_v, [idx])
                    if j == 0:
                        acc0 = acc0 + gathered
                    else:
                        acc1 = acc1 + gathered
                return (acc0, acc1)

            acc0, acc1 = quad_pair
            ev = en_v[pl.ds(c * CHUNK_COLS + lb, LANES)]
            row_v[pl.ds(lb, LANES)] = acc0 + acc1 + ev

        pltpu.sync_copy(row_v, out_hbm.at[pl.ds(col0, CHUNK_COLS)])
        wb.wait()


@jax.jit
def _run(spT, energies, tbl):
    mesh = plsc.VectorSubcoreMesh(core_axis_name="c", subcore_axis_name="s")
    return pl.kernel(
        _sc_body,
        mesh=mesh,
        compiler_params=pltpu.CompilerParams(needs_layout_passes=False),
        out_type=(jax.ShapeDtypeStruct((N_ATOMS, N_ROWS), jnp.int32),
                  jax.ShapeDtypeStruct((N_ROWS,), jnp.float32)),
        scratch_types=[
            pltpu.VMEM((N_ATOMS, CHUNK_COLS), jnp.int32),
            pltpu.VMEM((N_ATOMS, CHUNK_COLS), jnp.int32),
            pltpu.VMEM((4096,), jnp.float32),
            pltpu.VMEM((COLS_PER_W,), jnp.float32),
            pltpu.VMEM((CHUNK_COLS,), jnp.float32),
            pltpu.SemaphoreType.DMA,
            pltpu.SemaphoreType.DMA,
            pltpu.SemaphoreType.DMA,
            pltpu.SemaphoreType.DMA,
            pltpu.SemaphoreType.DMA,
        ],
    )(spT, energies, tbl)


def kernel(species, energies, self_energies):
    se = self_energies.astype(jnp.float32)
    tbl = (se[:, None, None, None] + se[None, :, None, None]
           + se[None, None, :, None] + se[None, None, None, :]).reshape(4096)
    out_spT, shifted = _run(species.T, energies, tbl)
    return (out_spT.T, shifted)
